# baseline (speedup 1.0000x reference)
import functools

import jax
import jax.numpy as jnp
from jax import lax
from jax.experimental import pallas as pl
from jax.experimental.pallas import tpu as pltpu
from jax.experimental.pallas import tpu_sc as plsc

VOCAB = 100000
D = 128
NC = 2
NS = 16
NW = NC * NS


def _embed_lookup(idx_flat, weight, *, n_rows, chunk):
    b_per_w = n_rows // NW
    n_chunks = b_per_w // chunk
    assert n_chunks % 3 == 1 and n_chunks >= 7
    mesh = plsc.VectorSubcoreMesh(core_axis_name="c", subcore_axis_name="s")

    @functools.partial(
        pl.kernel,
        mesh=mesh,
        out_type=jax.ShapeDtypeStruct((n_rows, D), jnp.float32),
        scratch_types=(
            [pltpu.VMEM((chunk,), jnp.int32) for _ in range(3)]
            + [pltpu.VMEM((chunk, D), jnp.float32) for _ in range(3)]
            + [pltpu.SemaphoreType.DMA for _ in range(9)]
        ),
    )
    def k(idx_hbm, table_hbm, out_hbm, *refs):
        idx_v = refs[0:3]
        rows_v = refs[3:6]
        isem = refs[6:9]
        gsem = refs[9:12]
        ssem = refs[12:15]
        wid = lax.axis_index("s") * NC + lax.axis_index("c")
        base = wid * b_per_w

        def idx_start(c, b):
            off = base + c * chunk
            pltpu.async_copy(idx_hbm.at[pl.ds(off, chunk)], idx_v[b], isem[b])

        def idx_wait(b):
            pltpu.make_async_copy(idx_hbm.at[pl.ds(base, chunk)], idx_v[b],
                                  isem[b]).wait()

        def gather_start(b):
            pltpu.async_copy(table_hbm.at[idx_v[b]], rows_v[b], gsem[b])

        def gather_wait(b):
            pltpu.make_async_copy(table_hbm.at[idx_v[b]], rows_v[b],
                                  gsem[b]).wait()

        def store_start(c, b):
            off = base + c * chunk
            pltpu.async_copy(rows_v[b], out_hbm.at[pl.ds(off, chunk)],
                             ssem[b])

        def store_wait(c, b):
            off = base + c * chunk
            pltpu.make_async_copy(rows_v[b], out_hbm.at[pl.ds(off, chunk)],
                                  ssem[b]).wait()

        for b in range(3):
            idx_start(b, b)
        idx_wait(0)
        gather_start(0)
        gather_wait(0)
        store_start(0, 0)
        idx_start(3, 0)
        idx_wait(1)
        gather_start(1)
        gather_wait(1)
        store_start(1, 1)
        idx_start(4, 1)
        idx_wait(2)
        gather_start(2)

        @pl.loop(2, n_chunks - 2, step=3)
        def _(c0):
            for ph in range(3):
                c = c0 + ph
                b = (2 + ph) % 3
                bn = (b + 1) % 3
                gather_wait(b)
                store_start(c, b)

                @pl.when(c + 3 < n_chunks)
                def _():
                    idx_start(c + 3, b)

                store_wait(c - 2, bn)
                idx_wait(bn)
                gather_start(bn)

        c = n_chunks - 2
        b = c % 3
        bn = (b + 1) % 3
        gather_wait(b)
        store_start(c, b)
        store_wait(c - 2, bn)
        idx_wait(bn)
        gather_start(bn)
        gather_wait(bn)
        store_start(c + 1, bn)
        store_wait(c - 1, (bn + 1) % 3)
        store_wait(c, b)
        store_wait(c + 1, bn)

    return k(idx_flat, weight)


def kernel(inputs, weight):
    b, t = inputs.shape
    n_rows = b * t
    idx_flat = inputs.reshape(n_rows).astype(jnp.int32)
    out = _embed_lookup(idx_flat, weight, n_rows=n_rows, chunk=256)
    return out.reshape(b, t, D)


# speedup vs baseline: 1.0117x; 1.0117x over previous
import functools

import jax
import jax.numpy as jnp
from jax import lax
from jax.experimental import pallas as pl
from jax.experimental.pallas import tpu as pltpu
from jax.experimental.pallas import tpu_sc as plsc

VOCAB = 100000
D = 128
NC = 2
NS = 16
NW = NC * NS


def _embed_lookup(idx_flat, weight, *, n_rows, chunk):
    b_per_w = n_rows // NW
    n_chunks = b_per_w // chunk
    assert n_chunks % 2 == 0 and n_chunks >= 4
    mesh = plsc.VectorSubcoreMesh(core_axis_name="c", subcore_axis_name="s")

    @functools.partial(
        pl.kernel,
        mesh=mesh,
        out_type=jax.ShapeDtypeStruct((n_rows, D), jnp.float32),
        scratch_types=[
            pltpu.VMEM((chunk,), jnp.int32),
            pltpu.VMEM((chunk,), jnp.int32),
            pltpu.VMEM((chunk, D), jnp.float32),
            pltpu.VMEM((chunk, D), jnp.float32),
            pltpu.SemaphoreType.DMA,
            pltpu.SemaphoreType.DMA,
            pltpu.SemaphoreType.DMA,
            pltpu.SemaphoreType.DMA,
        ],
    )
    def k(idx_hbm, table_hbm, out_hbm, i0, i1, r0, r1, g0, g1, s0, s1):
        wid = lax.axis_index("s") * NC + lax.axis_index("c")
        base = wid * b_per_w
        idx_v = (i0, i1)
        rows_v = (r0, r1)
        gsem = (g0, g1)
        ssem = (s0, s1)

        def gather_start(ci, b):
            off = base + ci * chunk
            pltpu.sync_copy(idx_hbm.at[pl.ds(off, chunk)], idx_v[b])
            pltpu.async_copy(table_hbm.at[idx_v[b]], rows_v[b], gsem[b])

        def gather_wait(b):
            pltpu.make_async_copy(table_hbm.at[idx_v[b]], rows_v[b],
                                  gsem[b]).wait()

        def store_start(ci, b):
            off = base + ci * chunk
            pltpu.async_copy(rows_v[b], out_hbm.at[pl.ds(off, chunk)],
                             ssem[b])

        def store_wait(ci, b):
            off = base + ci * chunk
            pltpu.make_async_copy(rows_v[b], out_hbm.at[pl.ds(off, chunk)],
                                  ssem[b]).wait()

        gather_start(0, 0)
        gather_start(1, 1)

        @pl.loop(0, n_chunks - 2, step=2)
        def _(ci):
            for ph in range(2):
                c = ci + ph
                gather_wait(ph)
                store_start(c, ph)
                store_wait(c, ph)
                gather_start(c + 2, ph)

        for ph in range(2):
            c = n_chunks - 2 + ph
            gather_wait(ph)
            store_start(c, ph)
        for ph in range(2):
            store_wait(n_chunks - 2 + ph, ph)

    return k(idx_flat, weight)


def kernel(inputs, weight):
    b, t = inputs.shape
    n_rows = b * t
    idx_flat = inputs.reshape(n_rows).astype(jnp.int32)
    out = _embed_lookup(idx_flat, weight, n_rows=n_rows, chunk=400)
    return out.reshape(b, t, D)


# speedup vs baseline: 1.0161x; 1.0043x over previous
import functools

import jax
import jax.numpy as jnp
from jax import lax
from jax.experimental import pallas as pl
from jax.experimental.pallas import tpu as pltpu
from jax.experimental.pallas import tpu_sc as plsc

VOCAB = 100000
D = 128
NC = 2
NS = 16
NW = NC * NS


def _embed_lookup(idx_flat, weight, *, n_rows, chunk):
    b_per_w = n_rows // NW
    n_chunks = b_per_w // chunk
    assert (n_chunks - 4) % 4 == 0 and n_chunks >= 8
    mesh = plsc.VectorSubcoreMesh(core_axis_name="c", subcore_axis_name="s")

    @functools.partial(
        pl.kernel,
        mesh=mesh,
        out_type=jax.ShapeDtypeStruct((n_rows, D), jnp.float32),
        scratch_types=(
            [pltpu.VMEM((chunk,), jnp.int32) for _ in range(4)]
            + [pltpu.VMEM((chunk, D), jnp.float32) for _ in range(4)]
            + [pltpu.SemaphoreType.DMA for _ in range(12)]
        ),
    )
    def k(idx_hbm, table_hbm, out_hbm, *refs):
        idx_v = refs[0:4]
        rows_v = refs[4:8]
        isem = refs[8:12]
        gsem = refs[12:16]
        ssem = refs[16:20]
        wid = lax.axis_index("s") * NC + lax.axis_index("c")
        base = wid * b_per_w

        def idx_start(c, b):
            off = base + c * chunk
            pltpu.async_copy(idx_hbm.at[pl.ds(off, chunk)], idx_v[b], isem[b])

        def idx_wait(b):
            pltpu.make_async_copy(idx_hbm.at[pl.ds(base, chunk)], idx_v[b],
                                  isem[b]).wait()

        def gather_start(b):
            pltpu.async_copy(table_hbm.at[idx_v[b]], rows_v[b], gsem[b])

        def gather_wait(b):
            pltpu.make_async_copy(table_hbm.at[idx_v[b]], rows_v[b],
                                  gsem[b]).wait()

        def store_start(c, b):
            off = base + c * chunk
            pltpu.async_copy(rows_v[b], out_hbm.at[pl.ds(off, chunk)],
                             ssem[b])

        def store_wait(c, b):
            off = base + c * chunk
            pltpu.make_async_copy(rows_v[b], out_hbm.at[pl.ds(off, chunk)],
                                  ssem[b]).wait()

        idx_start(0, 0)
        idx_start(1, 1)
        idx_start(2, 2)
        idx_wait(0)
        gather_start(0)
        idx_wait(1)
        gather_start(1)
        idx_wait(2)
        gather_start(2)
        gather_wait(0)
        store_start(0, 0)
        idx_start(3, 3)
        idx_wait(3)
        gather_start(3)
        gather_wait(1)
        store_start(1, 1)
        idx_start(4, 0)

        @pl.loop(2, n_chunks - 2, step=4)
        def _(c0):
            for ph in range(4):
                c = c0 + ph
                b = (2 + ph) % 4
                b2 = (b + 2) % 4
                b3 = (b + 3) % 4
                store_wait(c - 2, b2)
                idx_wait(b2)
                gather_start(b2)
                gather_wait(b)
                store_start(c, b)

                @pl.when(c + 3 < n_chunks)
                def _():
                    idx_start(c + 3, b3)

        gather_wait(2)
        store_start(n_chunks - 2, 2)
        gather_wait(3)
        store_start(n_chunks - 1, 3)
        store_wait(n_chunks - 4, 0)
        store_wait(n_chunks - 3, 1)
        store_wait(n_chunks - 2, 2)
        store_wait(n_chunks - 1, 3)

    return k(idx_flat, weight)


def kernel(inputs, weight):
    b, t = inputs.shape
    n_rows = b * t
    idx_flat = inputs.reshape(n_rows).astype(jnp.int32)
    out = _embed_lookup(idx_flat, weight, n_rows=n_rows, chunk=200)
    return out.reshape(b, t, D)
